# Initial kernel scaffold; baseline (speedup 1.0000x reference)
import functools

import jax
import jax.numpy as jnp
from jax import lax
from jax.experimental import pallas as pl
from jax.experimental.pallas import tpu as pltpu
from jax.experimental.pallas import tpu_sc as plsc

N_CLS = 100
CTX_DIM = 1024
EMBED_DIM = 512
BATCH = 16384

_PAD_CLS = 128
_NC, _NS = 2, 16
_NW = _NC * _NS
_B_PER_W = BATCH // _NW
_CHUNK = 128
_N_CHUNKS = _B_PER_W // _CHUNK


def _table_body(base_ref, ctx_ref, w_ref, b_ref, out_ref):
    proj = jnp.dot(ctx_ref[...], w_ref[...], preferred_element_type=jnp.float32)
    feat = base_ref[...] + proj + b_ref[...]
    ss = jnp.sum(feat * feat, axis=1, keepdims=True)
    out_ref[...] = feat * lax.rsqrt(ss)


def _build_table(base_p, ctx_p, w, b2d):
    return pl.pallas_call(
        _table_body,
        out_shape=jax.ShapeDtypeStruct((_PAD_CLS, EMBED_DIM), jnp.float32),
    )(base_p, ctx_p, w, b2d)


def _gather_body(table_hbm, idx_hbm, out_hbm, idx_v, rows_v, sem):
    wid = lax.axis_index("s") * _NC + lax.axis_index("c")
    pltpu.sync_copy(idx_hbm.at[wid], idx_v)
    for j in range(_N_CHUNKS):
        pltpu.async_copy(table_hbm.at[idx_v.at[j]], rows_v, sem).wait()
        row0 = wid * _B_PER_W + j * _CHUNK
        pltpu.sync_copy(rows_v, out_hbm.at[pl.ds(row0, _CHUNK)])


_gather = functools.partial(
    pl.kernel,
    mesh=plsc.VectorSubcoreMesh(core_axis_name="c", subcore_axis_name="s"),
    out_type=jax.ShapeDtypeStruct((BATCH, EMBED_DIM), jnp.float32),
    scratch_types=[
        pltpu.VMEM((_N_CHUNKS, _CHUNK), jnp.int32),
        pltpu.VMEM((_CHUNK, EMBED_DIM), jnp.float32),
        pltpu.SemaphoreType.DMA,
    ],
)(_gather_body)


def kernel(class_indices, base_features, prompt_ctx, W, b):
    pad = _PAD_CLS - N_CLS
    base_p = jnp.pad(base_features, ((0, pad), (0, 0)))
    ctx_p = jnp.pad(prompt_ctx, ((0, pad), (0, 0)))
    table = _build_table(base_p, ctx_p, W, b.reshape(1, EMBED_DIM))
    idx = class_indices.reshape(_NW, _N_CHUNKS, _CHUNK)
    return _gather(table, idx)


# speedup vs baseline: 3.1928x; 3.1928x over previous
import functools

import jax
import jax.numpy as jnp
from jax import lax
from jax.experimental import pallas as pl
from jax.experimental.pallas import tpu as pltpu
from jax.experimental.pallas import tpu_sc as plsc

N_CLS = 100
CTX_DIM = 1024
EMBED_DIM = 512
BATCH = 16384

_PAD_CLS = 128
_NC, _NS = 2, 16
_NW = _NC * _NS
_B_PER_W = BATCH // _NW
_CHUNK = 128
_N_CHUNKS = _B_PER_W // _CHUNK


def _table_body(base_ref, ctx_ref, w_ref, b_ref, out_ref):
    proj = jnp.dot(ctx_ref[...], w_ref[...], preferred_element_type=jnp.float32)
    feat = base_ref[...] + proj + b_ref[...]
    ss = jnp.sum(feat * feat, axis=1, keepdims=True)
    out_ref[...] = feat * lax.rsqrt(ss)


def _build_table(base_p, ctx_p, w, b2d):
    return pl.pallas_call(
        _table_body,
        out_shape=jax.ShapeDtypeStruct((_PAD_CLS, EMBED_DIM), jnp.float32),
    )(base_p, ctx_p, w, b2d)


def _gather_body(table_hbm, idx_hbm, out_hbm, idx_v, rows_v, sem):
    wid = lax.axis_index("s") * _NC + lax.axis_index("c")
    pltpu.sync_copy(idx_hbm.at[wid], idx_v)
    for j in range(_N_CHUNKS):
        pltpu.async_copy(table_hbm.at[idx_v.at[j]], rows_v, sem).wait()
        row0 = wid * _B_PER_W + j * _CHUNK
        pltpu.sync_copy(rows_v, out_hbm.at[pl.ds(row0, _CHUNK)])


@functools.lru_cache(maxsize=1)
def _make_gather():
    return pl.kernel(
        _gather_body,
        mesh=plsc.VectorSubcoreMesh(core_axis_name="c", subcore_axis_name="s"),
        out_type=jax.ShapeDtypeStruct((BATCH, EMBED_DIM), jnp.float32),
        scratch_types=[
            pltpu.VMEM((_N_CHUNKS, _CHUNK), jnp.int32),
            pltpu.VMEM((_CHUNK, EMBED_DIM), jnp.float32),
            pltpu.SemaphoreType.DMA,
        ],
    )


def kernel(class_indices, base_features, prompt_ctx, W, b):
    pad = _PAD_CLS - N_CLS
    base_p = jnp.pad(base_features, ((0, pad), (0, 0)))
    ctx_p = jnp.pad(prompt_ctx, ((0, pad), (0, 0)))
    table = _build_table(base_p, ctx_p, W, b.reshape(1, EMBED_DIM))
    idx = class_indices.reshape(_NW, _N_CHUNKS, _CHUNK)
    return _make_gather()(table, idx)
